# baseline (speedup 1.0000x reference)
import functools

import jax
import jax.numpy as jnp
from jax import lax
from jax.experimental import pallas as pl
from jax.experimental.pallas import tpu as pltpu
from jax.experimental.pallas import tpu_sc as plsc

N_NODES = 100000
N_FEATURES = 128
CHUNK = 128
CPW = 25
ROWS_PW = CPW * CHUNK
WSTRIDE = 3128
LAST_BASE = N_NODES - ROWS_PW
NBUF = 4


def _emb_kernel(idx_hbm, table_hbm, out_hbm, idx_v, table_sh,
                buf0, buf1, buf2, buf3,
                gsem0, gsem1, gsem2, gsem3,
                osem0, osem1, osem2, osem3):
    sid = lax.axis_index("s")
    wid = sid * 2 + lax.axis_index("c")
    base = jnp.minimum(wid * WSTRIDE, LAST_BASE)

    @pl.when(sid == 0)
    def _():
        pltpu.sync_copy(table_hbm, table_sh)

    pltpu.sync_copy(idx_hbm.at[pl.ds(base, ROWS_PW)], idx_v)
    plsc.subcore_barrier()

    bufs = (buf0, buf1, buf2, buf3)
    gsems = (gsem0, gsem1, gsem2, gsem3)
    osems = (osem0, osem1, osem2, osem3)

    def gather(t, b):
        pltpu.async_copy(
            table_sh.at[idx_v.at[pl.ds(t * CHUNK, CHUNK)]], bufs[b], gsems[b])

    def gwait(b):
        pltpu.make_async_copy(
            table_sh.at[idx_v.at[pl.ds(0, CHUNK)]], bufs[b], gsems[b]).wait()

    def outcopy(t, b):
        pltpu.async_copy(
            bufs[b], out_hbm.at[pl.ds(base + t * CHUNK, CHUNK)], osems[b])

    def owait(b):
        pltpu.make_async_copy(
            bufs[b], out_hbm.at[pl.ds(base, CHUNK)], osems[b]).wait()

    gather(0, 0)
    gather(1, 1)
    gwait(0)
    outcopy(0, 0)
    gather(2, 2)
    gwait(1)
    outcopy(1, 1)
    gather(3, 3)
    gwait(2)
    outcopy(2, 2)
    owait(0)
    gather(4, 0)
    gwait(3)
    outcopy(3, 3)
    owait(1)
    gather(5, 1)

    def quad(p, carry):
        for b in range(NBUF):
            t = p * NBUF + b
            gwait(b)
            outcopy(t, b)
            bn = (b + 2) % NBUF
            owait(bn)
            gather(jnp.minimum(t + 2, CPW - 1), bn)
        return carry

    lax.fori_loop(1, 6, quad, 0)

    gwait(0)
    outcopy(CPW - 1, 0)
    gwait(1)
    owait(2)
    owait(3)
    owait(0)


@jax.jit
def _emb(node_specie, table):
    mesh = plsc.VectorSubcoreMesh(core_axis_name="c", subcore_axis_name="s")
    f = functools.partial(
        pl.kernel,
        mesh=mesh,
        out_type=jax.ShapeDtypeStruct((N_NODES, N_FEATURES), jnp.float32),
        scratch_types=[
            pltpu.VMEM((ROWS_PW,), jnp.int32),
            pltpu.VMEM_SHARED((100, N_FEATURES), jnp.float32),
            pltpu.VMEM((CHUNK, N_FEATURES), jnp.float32),
            pltpu.VMEM((CHUNK, N_FEATURES), jnp.float32),
            pltpu.VMEM((CHUNK, N_FEATURES), jnp.float32),
            pltpu.VMEM((CHUNK, N_FEATURES), jnp.float32),
            pltpu.SemaphoreType.DMA,
            pltpu.SemaphoreType.DMA,
            pltpu.SemaphoreType.DMA,
            pltpu.SemaphoreType.DMA,
            pltpu.SemaphoreType.DMA,
            pltpu.SemaphoreType.DMA,
            pltpu.SemaphoreType.DMA,
            pltpu.SemaphoreType.DMA,
        ],
    )(_emb_kernel)
    return f(node_specie, table)


def kernel(node_specie, embeddings_0):
    table = embeddings_0.reshape(embeddings_0.shape[0], N_FEATURES)
    out = _emb(node_specie, table)
    return out.reshape(N_NODES, N_FEATURES, 1)


# speedup vs baseline: 1.0226x; 1.0226x over previous
import functools

import jax
import jax.numpy as jnp
from jax import lax
from jax.experimental import pallas as pl
from jax.experimental.pallas import tpu as pltpu
from jax.experimental.pallas import tpu_sc as plsc

N_SPECIES = 100
N_NODES = 100000
N_FEATURES = 128
GCHUNK = 128
CHUNK = 256
NCH = 12
ROWS_PW = NCH * CHUNK + GCHUNK
WSTRIDE = 3128
LAST_BASE = N_NODES - ROWS_PW
NBUF = 3


def _emb_kernel(idx_hbm, table_hbm, out_hbm, idx_v, table_sh,
                buf0, buf1, buf2,
                gsem0, gsem1, gsem2,
                osem0, osem1, osem2):
    sid = lax.axis_index("s")
    wid = sid * 2 + lax.axis_index("c")
    base = jnp.minimum(wid * WSTRIDE, LAST_BASE)

    @pl.when(sid == 0)
    def _():
        pltpu.sync_copy(table_hbm, table_sh)

    pltpu.sync_copy(idx_hbm.at[pl.ds(base, ROWS_PW)], idx_v)
    plsc.subcore_barrier()

    bufs = (buf0, buf1, buf2)
    gsems = (gsem0, gsem1, gsem2)
    osems = (osem0, osem1, osem2)

    def g2(t, b):
        for h in range(2):
            pltpu.async_copy(
                table_sh.at[idx_v.at[pl.ds(t * CHUNK + h * GCHUNK, GCHUNK)]],
                bufs[b].at[pl.ds(h * GCHUNK, GCHUNK)], gsems[b])

    def gw2(b):
        pltpu.make_async_copy(
            out_hbm.at[pl.ds(0, CHUNK)], bufs[b], gsems[b]).wait()

    def out(t, b):
        pltpu.async_copy(
            bufs[b], out_hbm.at[pl.ds(base + t * CHUNK, CHUNK)], osems[b])

    def ow(b):
        pltpu.make_async_copy(
            bufs[b], out_hbm.at[pl.ds(base, CHUNK)], osems[b]).wait()

    g2(0, 0)
    for t in range(NCH):
        b = t % NBUF
        gw2(b)
        out(t, b)
        bn = (t + 1) % NBUF
        if t + 1 <= NCH:
            if t + 1 >= NBUF:
                ow(bn)
            if t + 1 < NCH:
                g2(t + 1, bn)
            else:
                pltpu.async_copy(
                    table_sh.at[idx_v.at[pl.ds(NCH * CHUNK, GCHUNK)]],
                    bufs[bn].at[pl.ds(0, GCHUNK)], gsems[bn])

    pltpu.make_async_copy(
        out_hbm.at[pl.ds(0, GCHUNK)], bufs[0].at[pl.ds(0, GCHUNK)],
        gsems[0]).wait()
    pltpu.async_copy(
        bufs[0].at[pl.ds(0, GCHUNK)],
        out_hbm.at[pl.ds(base + NCH * CHUNK, GCHUNK)], osems[0])
    ow(1)
    ow(2)
    pltpu.make_async_copy(
        bufs[0].at[pl.ds(0, GCHUNK)], out_hbm.at[pl.ds(base, GCHUNK)],
        osems[0]).wait()


@jax.jit
def _emb(node_specie, table):
    mesh = plsc.VectorSubcoreMesh(core_axis_name="c", subcore_axis_name="s")
    f = functools.partial(
        pl.kernel,
        mesh=mesh,
        out_type=jax.ShapeDtypeStruct((N_NODES, N_FEATURES), jnp.float32),
        scratch_types=[
            pltpu.VMEM((ROWS_PW,), jnp.int32),
            pltpu.VMEM_SHARED((N_SPECIES, N_FEATURES), jnp.float32),
            pltpu.VMEM((CHUNK, N_FEATURES), jnp.float32),
            pltpu.VMEM((CHUNK, N_FEATURES), jnp.float32),
            pltpu.VMEM((CHUNK, N_FEATURES), jnp.float32),
            pltpu.SemaphoreType.DMA,
            pltpu.SemaphoreType.DMA,
            pltpu.SemaphoreType.DMA,
            pltpu.SemaphoreType.DMA,
            pltpu.SemaphoreType.DMA,
            pltpu.SemaphoreType.DMA,
        ],
    )(_emb_kernel)
    return f(node_specie, table)


def kernel(node_specie, embeddings_0):
    table = embeddings_0.reshape(embeddings_0.shape[0], N_FEATURES)
    out = _emb(node_specie, table)
    return out.reshape(N_NODES, N_FEATURES, 1)


# speedup vs baseline: 1.0368x; 1.0139x over previous
import functools

import jax
import jax.numpy as jnp
from jax import lax
from jax.experimental import pallas as pl
from jax.experimental.pallas import tpu as pltpu
from jax.experimental.pallas import tpu_sc as plsc

N_SPECIES = 100
N_NODES = 100000
N_FEATURES = 128
GCHUNK = 128
CHUNK = 320
NCH = 10
ROWS_PW = NCH * CHUNK
WSTRIDE = 3128
LAST_BASE = N_NODES - ROWS_PW
NBUF = 3
PAD_SPECIES = 104
STRIPE = 8


def _emb_kernel(idx_hbm, table_hbm, out_hbm, idx_v, table_sh,
                buf0, buf1, buf2,
                gsem0, gsem1, gsem2,
                osem0, osem1, osem2, isem):
    sid = lax.axis_index("s")
    wid = sid * 2 + lax.axis_index("c")
    base = jnp.minimum(wid * WSTRIDE, LAST_BASE)

    idx_copy = pltpu.make_async_copy(
        idx_hbm.at[pl.ds(base, ROWS_PW)], idx_v, isem)
    idx_copy.start()
    offs = jnp.minimum(sid * STRIPE, PAD_SPECIES - STRIPE)
    pltpu.sync_copy(table_hbm.at[pl.ds(offs, STRIPE)],
                    table_sh.at[pl.ds(offs, STRIPE)])
    plsc.subcore_barrier()
    idx_copy.wait()

    bufs = (buf0, buf1, buf2)
    gsems = (gsem0, gsem1, gsem2)
    osems = (osem0, osem1, osem2)

    def g3(t, b):
        for h, (o, w) in enumerate(((0, GCHUNK), (GCHUNK, GCHUNK),
                                    (2 * GCHUNK, CHUNK - 2 * GCHUNK))):
            pltpu.async_copy(
                table_sh.at[idx_v.at[pl.ds(t * CHUNK + o, w)]],
                bufs[b].at[pl.ds(o, w)], gsems[b])

    def gw(b):
        pltpu.make_async_copy(
            out_hbm.at[pl.ds(0, CHUNK)], bufs[b], gsems[b]).wait()

    def out(t, b):
        pltpu.async_copy(
            bufs[b], out_hbm.at[pl.ds(base + t * CHUNK, CHUNK)], osems[b])

    def ow(b):
        pltpu.make_async_copy(
            bufs[b], out_hbm.at[pl.ds(base, CHUNK)], osems[b]).wait()

    g3(0, 0)
    for t in range(NCH):
        b = t % NBUF
        gw(b)
        out(t, b)
        bn = (t + 1) % NBUF
        if t + 1 >= NBUF:
            ow(bn)
        if t + 1 < NCH:
            g3(t + 1, bn)

    ow(2)
    ow(0)


@jax.jit
def _emb(node_specie, table):
    mesh = plsc.VectorSubcoreMesh(core_axis_name="c", subcore_axis_name="s")
    f = functools.partial(
        pl.kernel,
        mesh=mesh,
        out_type=jax.ShapeDtypeStruct((N_NODES, N_FEATURES), jnp.float32),
        scratch_types=[
            pltpu.VMEM((ROWS_PW,), jnp.int32),
            pltpu.VMEM_SHARED((PAD_SPECIES, N_FEATURES), jnp.float32),
            pltpu.VMEM((CHUNK, N_FEATURES), jnp.float32),
            pltpu.VMEM((CHUNK, N_FEATURES), jnp.float32),
            pltpu.VMEM((CHUNK, N_FEATURES), jnp.float32),
            pltpu.SemaphoreType.DMA,
            pltpu.SemaphoreType.DMA,
            pltpu.SemaphoreType.DMA,
            pltpu.SemaphoreType.DMA,
            pltpu.SemaphoreType.DMA,
            pltpu.SemaphoreType.DMA,
            pltpu.SemaphoreType.DMA,
        ],
    )(_emb_kernel)
    return f(node_specie, table)


def kernel(node_specie, embeddings_0):
    table = embeddings_0.reshape(embeddings_0.shape[0], N_FEATURES)
    table = jnp.pad(table, ((0, PAD_SPECIES - N_SPECIES), (0, 0)))
    out = _emb(node_specie, table)
    return out.reshape(N_NODES, N_FEATURES, 1)


# speedup vs baseline: 1.0398x; 1.0029x over previous
import functools

import jax
import jax.numpy as jnp
from jax import lax
from jax.experimental import pallas as pl
from jax.experimental.pallas import tpu as pltpu
from jax.experimental.pallas import tpu_sc as plsc

N_SPECIES = 100
N_NODES = 100000
N_FEATURES = 128
GCHUNK = 128
CHUNK = 320
NCH = 10
ROWS_PW = NCH * CHUNK
WSTRIDE = 3128
LAST_BASE = N_NODES - ROWS_PW
NBUF = 3
PAD_SPECIES = 104
STRIPE = 8


def _emb_kernel(idx_hbm, table_hbm, out_hbm, idx_v, table_sh,
                buf0, buf1, buf2,
                gsem0, gsem1, gsem2,
                osem0, osem1, osem2, isem):
    sid = lax.axis_index("s")
    wid = sid * 2 + lax.axis_index("c")
    base = jnp.minimum(wid * WSTRIDE, LAST_BASE)

    idx_copy = pltpu.make_async_copy(
        idx_hbm.at[pl.ds(base, ROWS_PW)], idx_v, isem)
    idx_copy.start()
    offs = jnp.minimum(sid * STRIPE, PAD_SPECIES - STRIPE)
    pltpu.sync_copy(table_hbm.at[pl.ds(offs, STRIPE)],
                    table_sh.at[pl.ds(offs, STRIPE)])
    plsc.subcore_barrier()
    idx_copy.wait()

    bufs = (buf0, buf1, buf2)
    gsems = (gsem0, gsem1, gsem2)
    osems = (osem0, osem1, osem2)

    def g3(t, b):
        for h, (o, w) in enumerate(((0, GCHUNK), (GCHUNK, GCHUNK),
                                    (2 * GCHUNK, CHUNK - 2 * GCHUNK))):
            pltpu.async_copy(
                table_sh.at[idx_v.at[pl.ds(t * CHUNK + o, w)]],
                bufs[b].at[pl.ds(o, w)], gsems[b])

    def gw(b):
        pltpu.make_async_copy(
            out_hbm.at[pl.ds(0, CHUNK)], bufs[b], gsems[b]).wait()

    def out(t, b):
        pltpu.async_copy(
            bufs[b], out_hbm.at[pl.ds(base + t * CHUNK, CHUNK)], osems[b])

    def ow(b):
        pltpu.make_async_copy(
            bufs[b], out_hbm.at[pl.ds(base, CHUNK)], osems[b]).wait()

    g3(0, 0)
    for t in range(NCH):
        b = t % NBUF
        gw(b)
        out(t, b)
        bn = (t + 1) % NBUF
        if t + 1 >= NBUF:
            ow(bn)
        if t + 1 < NCH:
            g3(t + 1, bn)

    ow(2)
    ow(0)


@jax.jit
def _emb(node_specie, table):
    mesh = plsc.VectorSubcoreMesh(core_axis_name="c", subcore_axis_name="s")
    f = functools.partial(
        pl.kernel,
        mesh=mesh,
        out_type=jax.ShapeDtypeStruct((N_NODES, N_FEATURES), jnp.float32),
        scratch_types=[
            pltpu.VMEM((ROWS_PW,), jnp.int32),
            pltpu.VMEM_SHARED((PAD_SPECIES, N_FEATURES), jnp.float32),
            pltpu.VMEM((CHUNK, N_FEATURES), jnp.float32),
            pltpu.VMEM((CHUNK, N_FEATURES), jnp.float32),
            pltpu.VMEM((CHUNK, N_FEATURES), jnp.float32),
            pltpu.SemaphoreType.DMA,
            pltpu.SemaphoreType.DMA,
            pltpu.SemaphoreType.DMA,
            pltpu.SemaphoreType.DMA,
            pltpu.SemaphoreType.DMA,
            pltpu.SemaphoreType.DMA,
            pltpu.SemaphoreType.DMA,
        ],
        compiler_params=pltpu.CompilerParams(
            skip_device_barrier=True,
            disable_bounds_checks=True,
            disable_semaphore_checks=True,
        ),
    )(_emb_kernel)
    return f(node_specie, table)


def kernel(node_specie, embeddings_0):
    table = embeddings_0.reshape(embeddings_0.shape[0], N_FEATURES)
    table = jnp.pad(table, ((0, PAD_SPECIES - N_SPECIES), (0, 0)))
    out = _emb(node_specie, table)
    return out.reshape(N_NODES, N_FEATURES, 1)


# speedup vs baseline: 1.0402x; 1.0004x over previous
import functools

import jax
import jax.numpy as jnp
from jax import lax
from jax.experimental import pallas as pl
from jax.experimental.pallas import tpu as pltpu
from jax.experimental.pallas import tpu_sc as plsc

N_SPECIES = 100
N_NODES = 100000
N_FEATURES = 128
GCHUNK = 128
CHUNK = 320
NCH = 10
ROWS_PW = NCH * CHUNK
WSTRIDE = 3128
LAST_BASE = N_NODES - ROWS_PW
NBUF = 3
STRIPE = 8
FULL_STRIPES = N_SPECIES // STRIPE


def _emb_kernel(idx_hbm, table_hbm, out_hbm, idx_v, table_sh,
                buf0, buf1, buf2,
                gsem0, gsem1, gsem2,
                osem0, osem1, osem2, isem):
    sid = lax.axis_index("s")
    wid = sid * 2 + lax.axis_index("c")
    base = jnp.minimum(wid * WSTRIDE, LAST_BASE)

    idx_copy = pltpu.make_async_copy(
        idx_hbm.at[pl.ds(base, ROWS_PW)], idx_v, isem)
    idx_copy.start()
    @pl.when(sid < FULL_STRIPES)
    def _():
        offs = pl.multiple_of(sid * STRIPE, STRIPE)
        pltpu.sync_copy(table_hbm.at[pl.ds(offs, STRIPE)],
                        table_sh.at[pl.ds(offs, STRIPE)])

    @pl.when(sid == FULL_STRIPES)
    def _():
        pltpu.sync_copy(
            table_hbm.at[pl.ds(FULL_STRIPES * STRIPE,
                               N_SPECIES - FULL_STRIPES * STRIPE)],
            table_sh.at[pl.ds(FULL_STRIPES * STRIPE,
                              N_SPECIES - FULL_STRIPES * STRIPE)])
    plsc.subcore_barrier()
    idx_copy.wait()

    bufs = (buf0, buf1, buf2)
    gsems = (gsem0, gsem1, gsem2)
    osems = (osem0, osem1, osem2)

    def g3(t, b):
        for h, (o, w) in enumerate(((0, GCHUNK), (GCHUNK, GCHUNK),
                                    (2 * GCHUNK, CHUNK - 2 * GCHUNK))):
            pltpu.async_copy(
                table_sh.at[idx_v.at[pl.ds(t * CHUNK + o, w)]],
                bufs[b].at[pl.ds(o, w)], gsems[b])

    def gw(b):
        pltpu.make_async_copy(
            out_hbm.at[pl.ds(0, CHUNK)], bufs[b], gsems[b]).wait()

    def out(t, b):
        pltpu.async_copy(
            bufs[b], out_hbm.at[pl.ds(base + t * CHUNK, CHUNK)], osems[b])

    def ow(b):
        pltpu.make_async_copy(
            bufs[b], out_hbm.at[pl.ds(base, CHUNK)], osems[b]).wait()

    g3(0, 0)
    for t in range(NCH):
        b = t % NBUF
        gw(b)
        out(t, b)
        bn = (t + 1) % NBUF
        if t + 1 >= NBUF:
            ow(bn)
        if t + 1 < NCH:
            g3(t + 1, bn)

    ow(2)
    ow(0)


@jax.jit
def _emb(node_specie, table):
    mesh = plsc.VectorSubcoreMesh(core_axis_name="c", subcore_axis_name="s")
    f = functools.partial(
        pl.kernel,
        mesh=mesh,
        out_type=jax.ShapeDtypeStruct((N_NODES, N_FEATURES), jnp.float32),
        scratch_types=[
            pltpu.VMEM((ROWS_PW,), jnp.int32),
            pltpu.VMEM_SHARED((N_SPECIES, N_FEATURES), jnp.float32),
            pltpu.VMEM((CHUNK, N_FEATURES), jnp.float32),
            pltpu.VMEM((CHUNK, N_FEATURES), jnp.float32),
            pltpu.VMEM((CHUNK, N_FEATURES), jnp.float32),
            pltpu.SemaphoreType.DMA,
            pltpu.SemaphoreType.DMA,
            pltpu.SemaphoreType.DMA,
            pltpu.SemaphoreType.DMA,
            pltpu.SemaphoreType.DMA,
            pltpu.SemaphoreType.DMA,
            pltpu.SemaphoreType.DMA,
        ],
    )(_emb_kernel)
    return f(node_specie, table)


def kernel(node_specie, embeddings_0):
    table = embeddings_0.reshape(embeddings_0.shape[0], N_FEATURES)
    out = _emb(node_specie, table)
    return out.reshape(N_NODES, N_FEATURES, 1)
